# baseline (speedup 1.0000x reference)
import functools

import jax
import jax.numpy as jnp
from jax import lax
from jax.experimental import pallas as pl
from jax.experimental.pallas import tpu as pltpu
from jax.experimental.pallas import tpu_sc as plsc

N = 100000
M = 1000
D = 128
CH = 256
NC, NS = 2, 16
NW = NC * NS
FULL = N // CH
REM = N - FULL * CH
TPW = (FULL + NW - 1) // NW


def _sc_body(x_hbm, vals_hbm, idx_hbm, out_hbm, buf, idxb, tab):
    wid = lax.axis_index("s") * NC + lax.axis_index("c")
    pltpu.sync_copy(vals_hbm, tab)
    col128 = jnp.full((16,), D, jnp.int32)
    riota = lax.broadcasted_iota(jnp.int32, (16,), 0)

    def do_chunk(cid, rows):
        pltpu.sync_copy(idx_hbm.at[pl.ds(cid * CH, rows)], idxb.at[pl.ds(0, rows)])
        pltpu.sync_copy(x_hbm.at[pl.ds(cid * CH, rows), :], buf.at[pl.ds(0, rows), 0:D])
        for j in range(rows // 16):
            iv = idxb[pl.ds(j * 16, 16)]
            vals = plsc.load_gather(tab, [iv])
            plsc.store_scatter(buf, [riota + j * 16, col128], vals)
        pltpu.sync_copy(buf.at[pl.ds(0, rows), :], out_hbm.at[pl.ds(cid * CH, rows), :])

    for t in range(TPW):
        cid = wid + NW * t

        @pl.when(cid < FULL)
        def _full():
            do_chunk(cid, CH)

    @pl.when(wid == FULL % NW)
    def _rem():
        do_chunk(FULL, REM)


def kernel(per_atom_property_tensor, per_molecule_values, atomic_subsystem_indices):
    vals_p = jnp.zeros((1024,), jnp.float32).at[:M].set(per_molecule_values)
    mesh = plsc.VectorSubcoreMesh(
        core_axis_name="c", subcore_axis_name="s", num_cores=NC, num_subcores=NS)
    f = pl.kernel(
        _sc_body,
        out_type=jax.ShapeDtypeStruct((N, D + 1), jnp.float32),
        mesh=mesh,
        scratch_types=[
            pltpu.VMEM((CH, D + 1), jnp.float32),
            pltpu.VMEM((CH,), jnp.int32),
            pltpu.VMEM((1024,), jnp.float32),
        ],
        compiler_params=pltpu.CompilerParams(needs_layout_passes=False),
    )
    return f(per_atom_property_tensor, vals_p, atomic_subsystem_indices)
